# Initial kernel scaffold; baseline (speedup 1.0000x reference)
import jax
import jax.numpy as jnp
from jax.experimental import pallas as pl


def kernel(input_ids, offsets, weight):
    raise NotImplementedError("write your pallas kernel here")


# speedup vs baseline: 193.9740x; 193.9740x over previous
import jax
import jax.numpy as jnp
from jax import lax
from jax.experimental import pallas as pl
from jax.experimental.pallas import tpu as pltpu
from jax.experimental.pallas import tpu_sc as plsc

_VOCAB = 100000
_D = 128
_B = 16384
_N = 819200

_NC = 2
_NS = 16
_NW = _NC * _NS
_BAGS = _B // _NW
_K = 256
_ROWS = _BAGS + 8
_OFFV = _BAGS + 16


def _body(ids_hbm, off_hbm, w_hbm, out_hbm, off_v, ids_v, seg_v, rows_v,
          recip_v, shared):
    c = lax.axis_index("c")
    s = lax.axis_index("s")
    w = c * _NS + s
    b0 = w * _BAGS
    base = s * _ROWS
    iota = lax.iota(jnp.int32, 16)

    pltpu.sync_copy(off_hbm.at[pl.ds(b0, _OFFV)], off_v)

    def _zero(i, carry):
        for d in range(_D // 16):
            rows_v[i, pl.ds(d * 16, 16)] = jnp.zeros((16,), jnp.float32)
        return carry
    lax.fori_loop(0, _K, _zero, 0)
    pltpu.sync_copy(rows_v, shared.at[pl.ds(base, _K)])
    pltpu.sync_copy(rows_v, shared.at[pl.ds(base + _K, _K)])
    pltpu.sync_copy(rows_v.at[pl.ds(0, 8)], shared.at[pl.ds(base + _BAGS, 8)])

    t0 = off_v[pl.ds(0, 16)][0]
    t1 = off_v[pl.ds(_BAGS, 16)][0]
    p0 = (t0 // 8) * 8
    nchunks = (t1 - p0 + _K - 1) // _K

    def _chunk(i, carry):
        p = p0 + i * _K
        pltpu.sync_copy(ids_hbm.at[pl.ds(p, _K)], ids_v)
        for j in range(_K // 128):
            pltpu.sync_copy(w_hbm.at[ids_v.at[pl.ds(j * 128, 128)]],
                            rows_v.at[pl.ds(j * 128, 128)])

        def _seg(j, carry2):
            pos = p + j * 16 + iota
            lo = jnp.zeros((16,), jnp.int32)
            hi = jnp.full((16,), _BAGS + 1, jnp.int32)
            for _ in range(10):
                mid = (lo + hi) // 2
                val = plsc.load_gather(off_v, [mid])
                le = val <= pos
                lo = jnp.where(le, mid + 1, lo)
                hi = jnp.where(le, hi, mid)
            lid = jnp.where(lo == 0, _BAGS, lo - 1)
            seg_v[j // 8, pl.ds((j % 8) * 16, 16)] = base + lid
            return carry2
        lax.fori_loop(0, _K // 16, _seg, 0)

        for j in range(_K // 128):
            pltpu.sync_copy(rows_v.at[pl.ds(j * 128, 128)],
                            shared.at[seg_v.at[j]], add=True)
        return carry
    lax.fori_loop(0, nchunks, _chunk, 0)

    def _recip(j, carry):
        lo_v = off_v[pl.ds(j * 16, 16)]
        hi_v = off_v[pl.ds(j * 16 + 1, 16)]
        cnt = (hi_v - lo_v).astype(jnp.float32)
        recip_v[pl.ds(j * 16, 16)] = 1.0 / jnp.maximum(cnt, 1.0)
        return carry
    lax.fori_loop(0, _BAGS // 16, _recip, 0)

    for r in range(_BAGS // _K):
        pltpu.sync_copy(shared.at[pl.ds(base + r * _K, _K)], rows_v)

        def _div(lb, carry):
            splat = plsc.load_gather(
                recip_v, [jnp.zeros((16,), jnp.int32) + (lb + r * _K)])
            for d in range(_D // 16):
                rows_v[lb, pl.ds(d * 16, 16)] = (
                    rows_v[lb, pl.ds(d * 16, 16)] * splat)
            return carry
        lax.fori_loop(0, _K, _div, 0)
        pltpu.sync_copy(rows_v, out_hbm.at[pl.ds(b0 + r * _K, _K)])


_mesh = plsc.VectorSubcoreMesh(core_axis_name="c", subcore_axis_name="s")

_embed_bag = pl.kernel(
    _body,
    out_type=jax.ShapeDtypeStruct((_B, _D), jnp.float32),
    mesh=_mesh,
    scratch_types=[
        pltpu.VMEM((_OFFV,), jnp.int32),
        pltpu.VMEM((_K,), jnp.int32),
        pltpu.VMEM((_K // 128, 128), jnp.int32),
        pltpu.VMEM((_K, _D), jnp.float32),
        pltpu.VMEM((_BAGS,), jnp.float32),
        pltpu.VMEM_SHARED((_NS * _ROWS, _D), jnp.float32),
    ],
    compiler_params=pltpu.CompilerParams(needs_layout_passes=False),
)


@jax.jit
def kernel(input_ids, offsets, weight):
    ids = input_ids.astype(jnp.int32)
    off = offsets.astype(jnp.int32)
    pad_ids = (jnp.arange(2 * _K, dtype=jnp.int32) * 193) % _VOCAB
    ids_p = jnp.concatenate([ids, pad_ids])
    off_p = jnp.concatenate([off, jnp.full((16,), _N, jnp.int32)])
    return _embed_bag(ids_p, off_p, weight)
